# baseline (speedup 1.0000x reference)
import functools

import jax
import jax.numpy as jnp
from jax import lax
from jax.experimental import pallas as pl
from jax.experimental.pallas import tpu as pltpu
from jax.experimental.pallas import tpu_sc as plsc

_F = 26
_CT = 125
_NSLOT = 4
_CCS_E = 13
_CCS_O = 12
_NCH = 5


def _sc_body(x_hbm, z_hbm, out_hbm, xall, posstore, buf, sems):
    wid = lax.axis_index("c") * 16 + lax.axis_index("s")
    bp = wid >> 1
    parity = wid & 1
    lanes = lax.broadcasted_iota(jnp.int32, (16,), 0)
    ones16 = jnp.full((16,), 1.0, jnp.float32)
    zeros16 = jnp.zeros((16,), jnp.float32)

    for slot in range(_NSLOT):
        pltpu.sync_copy(z_hbm, buf.at[slot, pl.ds(0, 13)])
    pltpu.sync_copy(x_hbm.at[pl.ds(0, _F), bp], xall)

    def _scatter(slotv, p, val):
        plsc.store_scatter(buf, [slotv, p >> 11, (p >> 7) & 15, p & 127], val)

    def _make_loop(ccs, cbase):
        def _chunk(k, c):
            f = k // _NCH
            j = k - f * _NCH
            slot = k & (_NSLOT - 1)
            slotv = jnp.full((16,), slot, jnp.int32)
            cc0 = cbase + j * ccs

            @pl.when(k >= _NSLOT)
            def _retire():
                pltpu.make_async_copy(
                    buf.at[slot, pl.ds(0, ccs)],
                    out_hbm.at[0, pl.ds(0, ccs), 0],
                    sems.at[slot],
                ).wait()
                for v in range(16):
                    p = posstore[slot, pl.ds(v * 16, 16)]
                    _scatter(slotv, p, zeros16)

            for v in range(16):
                idx = xall[f, pl.ds(v * 16, 16)]
                cc = idx >> 3
                hit = (cc >= cc0) & (cc < cc0 + ccs)
                p = (((cc - cc0) << 11) + ((v >> 3) << 10)
                     + ((idx & 7) << 7) + (v & 7) * 16 + lanes)
                p = jnp.where(hit, p, 13 * 2048 + lanes)
                posstore[slot, pl.ds(v * 16, 16)] = p
                _scatter(slotv, p, ones16)

            pltpu.make_async_copy(
                buf.at[slot, pl.ds(0, ccs)],
                out_hbm.at[f, pl.ds(cc0, ccs), bp],
                sems.at[slot],
            ).start()
            return c

        lax.fori_loop(0, _F * _NCH, _chunk, 0)
        for slot in range(_NSLOT):
            pltpu.make_async_copy(
                buf.at[slot, pl.ds(0, ccs)],
                out_hbm.at[0, pl.ds(0, ccs), 0],
                sems.at[slot],
            ).wait()

    @pl.when(parity == 0)
    def _even():
        _make_loop(_CCS_E, 0)

    @pl.when(parity == 1)
    def _odd():
        _make_loop(_CCS_O, _CCS_E * _NCH)


def kernel(x):
    x = x.astype(jnp.int32)
    batch, feats = x.shape
    x_t3 = x.T.reshape(feats, 16, 256)
    zeros = jnp.zeros((13, 16, 128), jnp.float32)
    mesh = plsc.VectorSubcoreMesh(core_axis_name="c", subcore_axis_name="s")
    run = functools.partial(
        pl.kernel,
        mesh=mesh,
        out_type=jax.ShapeDtypeStruct((_F, _CT, 16, 16, 128), jnp.float32),
        compiler_params=pltpu.CompilerParams(needs_layout_passes=False),
        scratch_types=[
            pltpu.VMEM((_F, 256), jnp.int32),
            pltpu.VMEM((_NSLOT, 256), jnp.int32),
            pltpu.VMEM((_NSLOT, 14, 16, 128), jnp.float32),
            pltpu.SemaphoreType.DMA((_NSLOT,)),
        ],
    )(_sc_body)
    a = run(x_t3, zeros)
    a6 = a.reshape(_F, _CT, 16, 2, 8, 128)
    return a6.transpose(2, 3, 5, 0, 1, 4).reshape(batch, feats, 1000)
